# baseline (speedup 1.0000x reference)
import jax
import jax.numpy as jnp
from jax.experimental import pallas as pl
from jax.experimental.pallas import tpu as pltpu

K = 256
ROWS_PER_BLOCK = 32
NCOLS = 32768


def _body(x_ref, o_ref, key_ref, khi_ref):
    x = x_ref[...]
    i = jax.lax.bitcast_convert_type(x, jnp.int32)
    key = jnp.where(i >= 0, i, i ^ jnp.int32(0x7FFFFFFF))
    key_ref[...] = key
    sign = jnp.int32(-2147483648)
    kb_hi = jax.lax.shift_right_logical(key ^ sign, 16)
    khi_ref[...] = (kb_hi ^ 32768).astype(jnp.int16)

    NGROUPS = 8
    NTILES = NCOLS // 128

    def step_hi(b, tb_hi):
        bit = jax.lax.shift_left(jnp.int32(1), jnp.int32(15) - b)
        cand = tb_hi | bit
        cand16 = (cand - 32768).astype(jnp.int16)
        accs = [jnp.zeros((ROWS_PER_BLOCK, 128), jnp.int16)
                for _ in range(NGROUPS)]
        for i in range(NTILES):
            tile = khi_ref[:, i * 128:(i + 1) * 128]
            g = i % NGROUPS
            accs[g] = accs[g] + (tile >= cand16).astype(jnp.int16)
        while len(accs) > 1:
            accs = [accs[i] + accs[i + 1] for i in range(0, len(accs), 2)]
        cnt = jnp.sum(accs[0].astype(jnp.int32), axis=1, keepdims=True)
        return jnp.where(cnt >= K, cand, tb_hi)

    tb_hi = jax.lax.fori_loop(0, 16, step_hi,
                              jnp.zeros((ROWS_PER_BLOCK, 1), jnp.int32))

    def step_lo(b, tb):
        bit = jax.lax.shift_left(jnp.int32(1), jnp.int32(15) - b)
        candb = tb | bit
        cand = candb ^ sign
        accs = [jnp.zeros((ROWS_PER_BLOCK, 128), jnp.int32)
                for _ in range(NGROUPS)]
        for i in range(NTILES):
            tile = key_ref[:, i * 128:(i + 1) * 128]
            g = i % NGROUPS
            accs[g] = accs[g] + (tile >= cand).astype(jnp.int32)
        while len(accs) > 1:
            accs = [accs[i] + accs[i + 1] for i in range(0, len(accs), 2)]
        cnt = jnp.sum(accs[0], axis=1, keepdims=True)
        return jnp.where(cnt >= K, candb, tb)

    tb = jax.lax.fori_loop(0, 16, step_lo,
                           jax.lax.shift_left(tb_hi, 16))
    t = tb ^ sign
    o_ref[...] = jnp.where(key_ref[...] >= t, jnp.float32(0.0),
                           jnp.float32(jnp.inf))


def kernel(sim):
    nrows = sim.shape[0]
    grid = (nrows // ROWS_PER_BLOCK,)
    return pl.pallas_call(
        _body,
        grid=grid,
        in_specs=[pl.BlockSpec((ROWS_PER_BLOCK, NCOLS), lambda r: (r, 0))],
        out_specs=pl.BlockSpec((ROWS_PER_BLOCK, NCOLS), lambda r: (r, 0)),
        out_shape=jax.ShapeDtypeStruct(sim.shape, jnp.float32),
        scratch_shapes=[pltpu.VMEM((ROWS_PER_BLOCK, NCOLS), jnp.int32),
                        pltpu.VMEM((ROWS_PER_BLOCK, NCOLS), jnp.int16)],
    )(sim)


# speedup vs baseline: 1.0792x; 1.0792x over previous
import jax
import jax.numpy as jnp
from jax.experimental import pallas as pl
from jax.experimental.pallas import tpu as pltpu

K = 256
ROWS_PER_BLOCK = 64
NCOLS = 32768


def _body(x_ref, o_ref, key_ref):
    x = x_ref[...]
    i = jax.lax.bitcast_convert_type(x, jnp.int32)
    key = jnp.where(i >= 0, i, i ^ jnp.int32(0x7FFFFFFF))
    key_ref[...] = key

    sign = jnp.int32(-2147483648)
    NGROUPS = 8
    NTILES = NCOLS // 128

    def step(b, tb):
        bit = jax.lax.shift_left(jnp.int32(1), jnp.int32(31) - b)
        candb = tb | bit
        cand = candb ^ sign
        accs = [jnp.zeros((ROWS_PER_BLOCK, 128), jnp.int32)
                for _ in range(NGROUPS)]
        for i in range(NTILES):
            tile = key_ref[:, i * 128:(i + 1) * 128]
            g = i % NGROUPS
            accs[g] = accs[g] + (tile >= cand).astype(jnp.int32)
        while len(accs) > 1:
            accs = [accs[i] + accs[i + 1] for i in range(0, len(accs), 2)]
        cnt = jnp.sum(accs[0], axis=1, keepdims=True)
        return jnp.where(cnt >= K, candb, tb)

    tb0 = jnp.zeros((ROWS_PER_BLOCK, 1), jnp.int32)
    tb = jax.lax.fori_loop(0, 32, step, tb0)
    t = tb ^ sign
    o_ref[...] = jnp.where(key_ref[...] >= t, jnp.float32(0.0),
                           jnp.float32(jnp.inf))


def kernel(sim):
    nrows = sim.shape[0]
    grid = (nrows // ROWS_PER_BLOCK,)
    return pl.pallas_call(
        _body,
        grid=grid,
        in_specs=[pl.BlockSpec((ROWS_PER_BLOCK, NCOLS), lambda r: (r, 0))],
        out_specs=pl.BlockSpec((ROWS_PER_BLOCK, NCOLS), lambda r: (r, 0)),
        out_shape=jax.ShapeDtypeStruct(sim.shape, jnp.float32),
        scratch_shapes=[pltpu.VMEM((ROWS_PER_BLOCK, NCOLS), jnp.int32)],
    )(sim)
